# baseline (speedup 1.0000x reference)
import functools

import jax
import jax.numpy as jnp
from jax import lax
from jax.experimental import pallas as pl
from jax.experimental.pallas import tpu as pltpu
from jax.experimental.pallas import tpu_sc as plsc

T = 8192
D = 2048
OUT = 2048
E = 8
TOPK = 2

TB = 256
NTB = T // TB
BM = 256
CAP = TOPK * T + E * BM
NB = CAP // BM

NC = 2
NS = 16
NW = NC * NS
TPW = T // NW
CH = 16
NCH = TPW // CH


def _route_body(t_ref, nw_ref, nb_ref, rw_ref, rb_ref,
                x_ref, e0_ref, e1_ref, g0_ref, g1_ref, bh_ref):
    x = t_ref[...]
    mean = jnp.mean(x, axis=1, keepdims=True)
    xc = x - mean
    var = jnp.mean(xc * xc, axis=1, keepdims=True)
    xn = xc / jnp.sqrt(var + 1e-5)
    xn = xn * nw_ref[...] + nb_ref[...]
    x_ref[...] = xn.astype(jnp.bfloat16)
    logits = lax.dot_general(
        xn, rw_ref[...], (((1,), (1,)), ((), ())),
        preferred_element_type=jnp.float32) + rb_ref[...]
    idx = lax.broadcasted_iota(jnp.int32, (TB, E), 1)
    m0 = jnp.max(logits, axis=1, keepdims=True)
    e0 = jnp.min(jnp.where(logits == m0, idx, E), axis=1, keepdims=True)
    l2 = jnp.where(idx == e0, -jnp.inf, logits)
    m1 = jnp.max(l2, axis=1, keepdims=True)
    e1 = jnp.min(jnp.where(l2 == m1, idx, E), axis=1, keepdims=True)
    g0 = 1.0 / (1.0 + jnp.exp(m1 - m0))
    e0_ref[...] = e0
    e1_ref[...] = e1
    g0_ref[...] = jnp.broadcast_to(g0, (TB, 16))
    g1_ref[...] = jnp.broadcast_to(1.0 - g0, (TB, 16))
    oh = (idx == e0).astype(jnp.float32) + (idx == e1).astype(jnp.float32)
    bh_ref[...] = jnp.sum(oh, axis=0, keepdims=True)[None]


def _plan_body(bh_ref, e0_ref, e1_ref, pos0_ref, pos1_ref, bx_ref, carry):
    i = pl.program_id(0)

    @pl.when(i == 0)
    def _init():
        carry[...] = jnp.zeros((1, E), jnp.float32)

    totals = jnp.sum(bh_ref[...], axis=0)
    ps = jnp.ceil(totals / BM) * BM
    r8 = lax.broadcasted_iota(jnp.int32, (E, E), 0)
    c8 = lax.broadcasted_iota(jnp.int32, (E, E), 1)
    mexc = (r8 < c8).astype(jnp.float32)
    seg = lax.dot_general(ps, mexc, (((1,), (0,)), ((), ())),
                          preferred_element_type=jnp.float32)

    e0 = e0_ref[...]
    e1 = e1_ref[...]
    idx = lax.broadcasted_iota(jnp.int32, (TB, E), 1)
    oh0 = (idx == e0).astype(jnp.float32)
    oh1 = (idx == e1).astype(jnp.float32)
    oh = oh0 + oh1
    br = lax.broadcasted_iota(jnp.int32, (TB, TB), 0)
    bc = lax.broadcasted_iota(jnp.int32, (TB, TB), 1)
    tril = (br > bc).astype(jnp.float32)
    cexc = lax.dot_general(tril, oh, (((1,), (0,)), ((), ())),
                           preferred_element_type=jnp.float32)
    val = seg + carry[...] + cexc
    pos0_ref[...] = jnp.sum(jnp.where(idx == e0, val, 0.0), axis=1,
                            keepdims=True).astype(jnp.int32)
    pos1_ref[...] = jnp.sum(jnp.where(idx == e1, val, 0.0), axis=1,
                            keepdims=True).astype(jnp.int32)
    carry[...] += jnp.sum(oh, axis=0, keepdims=True)

    seg_end = seg + ps
    rnb = lax.broadcasted_iota(jnp.int32, (NB, E), 0).astype(jnp.float32)
    before = (seg_end <= rnb * BM).astype(jnp.int32)
    bx_ref[...] = jnp.minimum(jnp.sum(before, axis=1, keepdims=True), E - 1)


def _make_dispatch():
    mesh = plsc.VectorSubcoreMesh(core_axis_name="c", subcore_axis_name="s")

    @functools.partial(
        pl.kernel, mesh=mesh,
        out_type=jax.ShapeDtypeStruct((CAP, D // 2), jnp.float32),
        scratch_types=[
            pltpu.VMEM((CH,), jnp.int32),
            pltpu.VMEM((CH,), jnp.int32),
            pltpu.VMEM((CH, D // 2), jnp.float32),
            pltpu.SemaphoreType.DMA,
            pltpu.SemaphoreType.DMA,
        ],
    )
    def dispatch(x_hbm, pos0_hbm, pos1_hbm, xs_hbm,
                 idx0_v, idx1_v, rows_v, sem0, sem1):
        wid = lax.axis_index("s") * NC + lax.axis_index("c")

        def chunk(i, carry_):
            base = wid * TPW + i * CH
            pltpu.sync_copy(pos0_hbm.at[pl.ds(base, CH)], idx0_v)
            pltpu.sync_copy(pos1_hbm.at[pl.ds(base, CH)], idx1_v)
            pltpu.sync_copy(x_hbm.at[pl.ds(base, CH)], rows_v)
            cp0 = pltpu.async_copy(rows_v, xs_hbm.at[idx0_v], sem0)
            cp1 = pltpu.async_copy(rows_v, xs_hbm.at[idx1_v], sem1)
            cp0.wait()
            cp1.wait()
            return carry_

        lax.fori_loop(0, NCH, chunk, 0)

    return dispatch


def _group_mm_body(bx_ref, xs_ref, w_ref, b_ref, ys_ref):
    ys_ref[...] = lax.dot_general(
        xs_ref[...], w_ref[0], (((1,), (1,)), ((), ())),
        preferred_element_type=jnp.float32) + b_ref[0]


def _make_combine():
    mesh = plsc.VectorSubcoreMesh(core_axis_name="c", subcore_axis_name="s")
    KU = 8
    U = (OUT // 16) // KU

    @functools.partial(
        pl.kernel, mesh=mesh,
        out_type=jax.ShapeDtypeStruct((T, OUT), jnp.float32),
        scratch_types=[
            pltpu.VMEM((CH,), jnp.int32),
            pltpu.VMEM((CH,), jnp.int32),
            pltpu.VMEM((CH, 16), jnp.float32),
            pltpu.VMEM((CH, 16), jnp.float32),
            pltpu.VMEM((CH, OUT), jnp.float32),
            pltpu.VMEM((CH, OUT), jnp.float32),
            pltpu.VMEM((CH, OUT), jnp.float32),
            pltpu.SemaphoreType.DMA,
            pltpu.SemaphoreType.DMA,
        ],
    )
    def combine(ys_hbm, pos0_hbm, pos1_hbm, g0_hbm, g1_hbm, out_hbm,
                idx0_v, idx1_v, gv0_v, gv1_v, r0_v, r1_v, ob_v, sem0, sem1):
        wid = lax.axis_index("s") * NC + lax.axis_index("c")

        def chunk(i, carry_):
            base = wid * TPW + i * CH
            pltpu.sync_copy(pos0_hbm.at[pl.ds(base, CH)], idx0_v)
            pltpu.sync_copy(pos1_hbm.at[pl.ds(base, CH)], idx1_v)
            pltpu.sync_copy(g0_hbm.at[pl.ds(base, CH)], gv0_v)
            pltpu.sync_copy(g1_hbm.at[pl.ds(base, CH)], gv1_v)
            cp0 = pltpu.async_copy(ys_hbm.at[idx0_v], r0_v, sem0)
            cp1 = pltpu.async_copy(ys_hbm.at[idx1_v], r1_v, sem1)
            cp0.wait()
            cp1.wait()
            for j in range(CH):
                gj0 = gv0_v[j, :]
                gj1 = gv1_v[j, :]

                def krow(k, c, j=j, gj0=gj0, gj1=gj1):
                    for u in range(U):
                        sl = pl.ds((k * U + u) * 16, 16)
                        ob_v[j, sl] = gj0 * r0_v[j, sl] + gj1 * r1_v[j, sl]
                    return c

                lax.fori_loop(0, KU, krow, 0)
            pltpu.sync_copy(ob_v, out_hbm.at[pl.ds(base, CH)])
            return carry_

        lax.fori_loop(0, NCH, chunk, 0)

    return combine


_make_dispatch = functools.cache(_make_dispatch)
_make_combine = functools.cache(_make_combine)


def _route_call(tensor, norm_w, norm_b, router_w, router_b, interpret=False):
    return pl.pallas_call(
        _route_body,
        grid=(NTB,),
        in_specs=[
            pl.BlockSpec((TB, D), lambda i: (i, 0)),
            pl.BlockSpec((1, D), lambda i: (0, 0)),
            pl.BlockSpec((1, D), lambda i: (0, 0)),
            pl.BlockSpec((E, D), lambda i: (0, 0)),
            pl.BlockSpec((1, E), lambda i: (0, 0)),
        ],
        out_specs=[
            pl.BlockSpec((TB, D), lambda i: (i, 0)),
            pl.BlockSpec((TB, 1), lambda i: (i, 0)),
            pl.BlockSpec((TB, 1), lambda i: (i, 0)),
            pl.BlockSpec((TB, 16), lambda i: (i, 0)),
            pl.BlockSpec((TB, 16), lambda i: (i, 0)),
            pl.BlockSpec((1, 1, E), lambda i: (i, 0, 0)),
        ],
        out_shape=[
            jax.ShapeDtypeStruct((T, D), jnp.bfloat16),
            jax.ShapeDtypeStruct((T, 1), jnp.int32),
            jax.ShapeDtypeStruct((T, 1), jnp.int32),
            jax.ShapeDtypeStruct((T, 16), jnp.float32),
            jax.ShapeDtypeStruct((T, 16), jnp.float32),
            jax.ShapeDtypeStruct((NTB, 1, E), jnp.float32),
        ],
        interpret=interpret,
    )(tensor, norm_w.reshape(1, D), norm_b.reshape(1, D), router_w,
      router_b.reshape(1, E))


def _plan_call(bh, e0, e1, interpret=False):
    return pl.pallas_call(
        _plan_body,
        grid=(NTB,),
        in_specs=[
            pl.BlockSpec((NTB, 1, E), lambda i: (0, 0, 0)),
            pl.BlockSpec((TB, 1), lambda i: (i, 0)),
            pl.BlockSpec((TB, 1), lambda i: (i, 0)),
        ],
        out_specs=[
            pl.BlockSpec((TB, 1), lambda i: (i, 0)),
            pl.BlockSpec((TB, 1), lambda i: (i, 0)),
            pl.BlockSpec((NB, 1), lambda i: (0, 0)),
        ],
        out_shape=[
            jax.ShapeDtypeStruct((T, 1), jnp.int32),
            jax.ShapeDtypeStruct((T, 1), jnp.int32),
            jax.ShapeDtypeStruct((NB, 1), jnp.int32),
        ],
        scratch_shapes=[pltpu.VMEM((1, E), jnp.float32)],
        interpret=interpret,
    )(bh, e0, e1)


def _group_mm_call(bx, xs, expert_w, expert_b, interpret=False):
    return pl.pallas_call(
        _group_mm_body,
        grid_spec=pltpu.PrefetchScalarGridSpec(
            num_scalar_prefetch=1,
            grid=(NB,),
            in_specs=[
                pl.BlockSpec((BM, D), lambda b, bx: (b, 0)),
                pl.BlockSpec((1, OUT, D), lambda b, bx: (bx[b], 0, 0)),
                pl.BlockSpec((1, 1, OUT), lambda b, bx: (bx[b], 0, 0)),
            ],
            out_specs=pl.BlockSpec((BM, OUT), lambda b, bx: (b, 0)),
        ),
        out_shape=jax.ShapeDtypeStruct((CAP, OUT), jnp.float32),
        interpret=interpret,
    )(bx, xs, expert_w.astype(jnp.bfloat16), expert_b.reshape(E, 1, OUT))


@jax.jit
def kernel(tensor, norm_w, norm_b, router_w, router_b, expert_w, expert_b):
    x_norm, e0, e1, g0, g1, bh = _route_call(
        tensor, norm_w, norm_b, router_w, router_b)
    pos0, pos1, bx = _plan_call(bh, e0, e1)
    pos0f = pos0.reshape(T)
    pos1f = pos1.reshape(T)
    x_packed = lax.bitcast_convert_type(
        x_norm.reshape(T, D // 2, 2), jnp.float32)
    xs_packed = _make_dispatch()(x_packed, pos0f, pos1f)
    xs = lax.bitcast_convert_type(xs_packed, jnp.bfloat16).reshape(CAP, D)
    ys = _group_mm_call(bx.reshape(NB), xs, expert_w, expert_b)
    out = _make_combine()(ys, pos0f, pos1f, g0, g1)
    return out


# speedup vs baseline: 2.9513x; 2.9513x over previous
import functools

import jax
import jax.numpy as jnp
from jax import lax
from jax.experimental import pallas as pl
from jax.experimental.pallas import tpu as pltpu
from jax.experimental.pallas import tpu_sc as plsc

T = 8192
D = 2048
OUT = 2048
E = 8
TOPK = 2

TB = 256
NTB = T // TB
BM = 256
CAP = TOPK * T + E * BM
NB = CAP // BM

NC = 2
NS = 16
NW = NC * NS
TPW = T // NW
CH = 16
NCH = TPW // CH


def _route_body(t_ref, nw_ref, nb_ref, rw_ref, rb_ref,
                x_ref, e0_ref, e1_ref, g0_ref, g1_ref, bh_ref):
    x = t_ref[...]
    mean = jnp.mean(x, axis=1, keepdims=True)
    xc = x - mean
    var = jnp.mean(xc * xc, axis=1, keepdims=True)
    xn = xc / jnp.sqrt(var + 1e-5)
    xn = xn * nw_ref[...] + nb_ref[...]
    u_lo = lax.bitcast_convert_type(xn[:, :D // 2], jnp.uint32)
    u_hi = lax.bitcast_convert_type(xn[:, D // 2:], jnp.uint32)
    r_lo = (u_lo + 0x7FFF + ((u_lo >> 16) & 1)) >> 16
    r_hi = (u_hi + 0x7FFF + ((u_hi >> 16) & 1)) & jnp.uint32(0xFFFF0000)
    x_ref[...] = lax.bitcast_convert_type(r_lo | r_hi, jnp.float32)
    logits = lax.dot_general(
        xn, rw_ref[...], (((1,), (1,)), ((), ())),
        preferred_element_type=jnp.float32) + rb_ref[...]
    idx = lax.broadcasted_iota(jnp.int32, (TB, E), 1)
    m0 = jnp.max(logits, axis=1, keepdims=True)
    e0 = jnp.min(jnp.where(logits == m0, idx, E), axis=1, keepdims=True)
    l2 = jnp.where(idx == e0, -jnp.inf, logits)
    m1 = jnp.max(l2, axis=1, keepdims=True)
    e1 = jnp.min(jnp.where(l2 == m1, idx, E), axis=1, keepdims=True)
    g0 = 1.0 / (1.0 + jnp.exp(m1 - m0))
    e0_ref[...] = e0
    e1_ref[...] = e1
    g0_ref[...] = jnp.broadcast_to(g0, (TB, 16))
    g1_ref[...] = jnp.broadcast_to(1.0 - g0, (TB, 16))
    oh = (idx == e0).astype(jnp.float32) + (idx == e1).astype(jnp.float32)
    bh_ref[...] = jnp.sum(oh, axis=0, keepdims=True)[None]


def _plan_body(bh_ref, e0_ref, e1_ref, pos0_ref, pos1_ref, bx_ref, carry):
    i = pl.program_id(0)

    @pl.when(i == 0)
    def _init():
        carry[...] = jnp.zeros((1, E), jnp.float32)

    totals = jnp.sum(bh_ref[...], axis=0)
    ps = jnp.ceil(totals / BM) * BM
    r8 = lax.broadcasted_iota(jnp.int32, (E, E), 0)
    c8 = lax.broadcasted_iota(jnp.int32, (E, E), 1)
    mexc = (r8 < c8).astype(jnp.float32)
    seg = lax.dot_general(ps, mexc, (((1,), (0,)), ((), ())),
                          preferred_element_type=jnp.float32)

    e0 = e0_ref[...]
    e1 = e1_ref[...]
    idx = lax.broadcasted_iota(jnp.int32, (TB, E), 1)
    oh0 = (idx == e0).astype(jnp.float32)
    oh1 = (idx == e1).astype(jnp.float32)
    oh = oh0 + oh1
    br = lax.broadcasted_iota(jnp.int32, (TB, TB), 0)
    bc = lax.broadcasted_iota(jnp.int32, (TB, TB), 1)
    tril = (br > bc).astype(jnp.float32)
    cexc = lax.dot_general(tril, oh, (((1,), (0,)), ((), ())),
                           preferred_element_type=jnp.float32)
    val = seg + carry[...] + cexc
    pos0_ref[...] = jnp.sum(jnp.where(idx == e0, val, 0.0), axis=1,
                            keepdims=True).astype(jnp.int32)
    pos1_ref[...] = jnp.sum(jnp.where(idx == e1, val, 0.0), axis=1,
                            keepdims=True).astype(jnp.int32)
    carry[...] += jnp.sum(oh, axis=0, keepdims=True)

    seg_end = seg + ps
    rnb = lax.broadcasted_iota(jnp.int32, (NB, E), 0).astype(jnp.float32)
    before = (seg_end <= rnb * BM).astype(jnp.int32)
    bx_ref[...] = jnp.minimum(jnp.sum(before, axis=1, keepdims=True), E - 1)


def _make_dispatch():
    mesh = plsc.VectorSubcoreMesh(core_axis_name="c", subcore_axis_name="s")

    @functools.partial(
        pl.kernel, mesh=mesh,
        out_type=jax.ShapeDtypeStruct((CAP, D // 2), jnp.float32),
        scratch_types=[
            pltpu.VMEM((CH,), jnp.int32),
            pltpu.VMEM((CH,), jnp.int32),
            pltpu.VMEM((CH, D // 2), jnp.float32),
            pltpu.SemaphoreType.DMA,
            pltpu.SemaphoreType.DMA,
        ],
    )
    def dispatch(x_hbm, pos0_hbm, pos1_hbm, xs_hbm,
                 idx0_v, idx1_v, rows_v, sem0, sem1):
        wid = lax.axis_index("s") * NC + lax.axis_index("c")

        def chunk(i, carry_):
            base = wid * TPW + i * CH
            pltpu.sync_copy(pos0_hbm.at[pl.ds(base, CH)], idx0_v)
            pltpu.sync_copy(pos1_hbm.at[pl.ds(base, CH)], idx1_v)
            pltpu.sync_copy(x_hbm.at[pl.ds(base, CH)], rows_v)
            cp0 = pltpu.async_copy(rows_v, xs_hbm.at[idx0_v], sem0)
            cp1 = pltpu.async_copy(rows_v, xs_hbm.at[idx1_v], sem1)
            cp0.wait()
            cp1.wait()
            return carry_

        lax.fori_loop(0, NCH, chunk, 0)

    return dispatch


def _group_mm_body(bx_ref, xs_ref, w_ref, b_ref, ys_ref):
    w = lax.bitcast_convert_type(xs_ref[...], jnp.uint32)
    x_lo = lax.bitcast_convert_type(w << 16, jnp.float32)
    x_hi = lax.bitcast_convert_type(w & jnp.uint32(0xFFFF0000), jnp.float32)
    acc = lax.dot_general(
        x_lo, w_ref[0, :, 0:D // 2], (((1,), (1,)), ((), ())),
        preferred_element_type=jnp.float32)
    acc += lax.dot_general(
        x_hi, w_ref[0, :, D // 2:D], (((1,), (1,)), ((), ())),
        preferred_element_type=jnp.float32)
    ys_ref[...] = acc + b_ref[0]


def _make_combine():
    mesh = plsc.VectorSubcoreMesh(core_axis_name="c", subcore_axis_name="s")
    KU = 8
    U = (OUT // 16) // KU

    @functools.partial(
        pl.kernel, mesh=mesh,
        out_type=jax.ShapeDtypeStruct((T, OUT), jnp.float32),
        scratch_types=[
            pltpu.VMEM((CH,), jnp.int32),
            pltpu.VMEM((CH,), jnp.int32),
            pltpu.VMEM((CH, 16), jnp.float32),
            pltpu.VMEM((CH, 16), jnp.float32),
            pltpu.VMEM((CH, OUT), jnp.float32),
            pltpu.VMEM((CH, OUT), jnp.float32),
            pltpu.VMEM((CH, OUT), jnp.float32),
            pltpu.SemaphoreType.DMA,
            pltpu.SemaphoreType.DMA,
        ],
    )
    def combine(ys_hbm, pos0_hbm, pos1_hbm, g0_hbm, g1_hbm, out_hbm,
                idx0_v, idx1_v, gv0_v, gv1_v, r0_v, r1_v, ob_v, sem0, sem1):
        wid = lax.axis_index("s") * NC + lax.axis_index("c")

        def chunk(i, carry_):
            base = wid * TPW + i * CH
            pltpu.sync_copy(pos0_hbm.at[pl.ds(base, CH)], idx0_v)
            pltpu.sync_copy(pos1_hbm.at[pl.ds(base, CH)], idx1_v)
            pltpu.sync_copy(g0_hbm.at[pl.ds(base, CH)], gv0_v)
            pltpu.sync_copy(g1_hbm.at[pl.ds(base, CH)], gv1_v)
            cp0 = pltpu.async_copy(ys_hbm.at[idx0_v], r0_v, sem0)
            cp1 = pltpu.async_copy(ys_hbm.at[idx1_v], r1_v, sem1)
            cp0.wait()
            cp1.wait()
            for j in range(CH):
                gj0 = gv0_v[j, :]
                gj1 = gv1_v[j, :]

                def krow(k, c, j=j, gj0=gj0, gj1=gj1):
                    for u in range(U):
                        sl = pl.ds((k * U + u) * 16, 16)
                        ob_v[j, sl] = gj0 * r0_v[j, sl] + gj1 * r1_v[j, sl]
                    return c

                lax.fori_loop(0, KU, krow, 0)
            pltpu.sync_copy(ob_v, out_hbm.at[pl.ds(base, CH)])
            return carry_

        lax.fori_loop(0, NCH, chunk, 0)

    return combine


_make_dispatch = functools.cache(_make_dispatch)
_make_combine = functools.cache(_make_combine)


def _route_call(tensor, norm_w, norm_b, router_w, router_b, interpret=False):
    return pl.pallas_call(
        _route_body,
        grid=(NTB,),
        in_specs=[
            pl.BlockSpec((TB, D), lambda i: (i, 0)),
            pl.BlockSpec((1, D), lambda i: (0, 0)),
            pl.BlockSpec((1, D), lambda i: (0, 0)),
            pl.BlockSpec((E, D), lambda i: (0, 0)),
            pl.BlockSpec((1, E), lambda i: (0, 0)),
        ],
        out_specs=[
            pl.BlockSpec((TB, D // 2), lambda i: (i, 0)),
            pl.BlockSpec((TB, 1), lambda i: (i, 0)),
            pl.BlockSpec((TB, 1), lambda i: (i, 0)),
            pl.BlockSpec((TB, 16), lambda i: (i, 0)),
            pl.BlockSpec((TB, 16), lambda i: (i, 0)),
            pl.BlockSpec((1, 1, E), lambda i: (i, 0, 0)),
        ],
        out_shape=[
            jax.ShapeDtypeStruct((T, D // 2), jnp.float32),
            jax.ShapeDtypeStruct((T, 1), jnp.int32),
            jax.ShapeDtypeStruct((T, 1), jnp.int32),
            jax.ShapeDtypeStruct((T, 16), jnp.float32),
            jax.ShapeDtypeStruct((T, 16), jnp.float32),
            jax.ShapeDtypeStruct((NTB, 1, E), jnp.float32),
        ],
        interpret=interpret,
    )(tensor, norm_w.reshape(1, D), norm_b.reshape(1, D), router_w,
      router_b.reshape(1, E))


def _plan_call(bh, e0, e1, interpret=False):
    return pl.pallas_call(
        _plan_body,
        grid=(NTB,),
        in_specs=[
            pl.BlockSpec((NTB, 1, E), lambda i: (0, 0, 0)),
            pl.BlockSpec((TB, 1), lambda i: (i, 0)),
            pl.BlockSpec((TB, 1), lambda i: (i, 0)),
        ],
        out_specs=[
            pl.BlockSpec((TB, 1), lambda i: (i, 0)),
            pl.BlockSpec((TB, 1), lambda i: (i, 0)),
            pl.BlockSpec((NB, 1), lambda i: (0, 0)),
        ],
        out_shape=[
            jax.ShapeDtypeStruct((T, 1), jnp.int32),
            jax.ShapeDtypeStruct((T, 1), jnp.int32),
            jax.ShapeDtypeStruct((NB, 1), jnp.int32),
        ],
        scratch_shapes=[pltpu.VMEM((1, E), jnp.float32)],
        interpret=interpret,
    )(bh, e0, e1)


def _group_mm_call(bx, xs, expert_w, expert_b, interpret=False):
    return pl.pallas_call(
        _group_mm_body,
        grid_spec=pltpu.PrefetchScalarGridSpec(
            num_scalar_prefetch=1,
            grid=(NB,),
            in_specs=[
                pl.BlockSpec((BM, D // 2), lambda b, bx: (b, 0)),
                pl.BlockSpec((1, OUT, D), lambda b, bx: (bx[b], 0, 0)),
                pl.BlockSpec((1, 1, OUT), lambda b, bx: (bx[b], 0, 0)),
            ],
            out_specs=pl.BlockSpec((BM, OUT), lambda b, bx: (b, 0)),
        ),
        out_shape=jax.ShapeDtypeStruct((CAP, OUT), jnp.float32),
        interpret=interpret,
    )(bx, xs, expert_w, expert_b.reshape(E, 1, OUT))


@jax.jit
def kernel(tensor, norm_w, norm_b, router_w, router_b, expert_w, expert_b):
    x_norm, e0, e1, g0, g1, bh = _route_call(
        tensor, norm_w, norm_b, router_w, router_b)
    pos0, pos1, bx = _plan_call(bh, e0, e1)
    pos0f = pos0.reshape(T)
    pos1f = pos1.reshape(T)
    xs = _make_dispatch()(x_norm, pos0f, pos1f)
    ys = _group_mm_call(bx.reshape(NB), xs, expert_w, expert_b)
    out = _make_combine()(ys, pos0f, pos1f, g0, g1)
    return out
